# Initial kernel scaffold; baseline (speedup 1.0000x reference)
import jax
import jax.numpy as jnp
from jax.experimental import pallas as pl


def kernel(q, k, v):
    raise NotImplementedError("write your pallas kernel here")


# speedup vs baseline: 4.2244x; 4.2244x over previous
import math

import jax
import jax.numpy as jnp
import numpy as np
from jax.experimental import pallas as pl
from jax.experimental.pallas import tpu as pltpu

B = 4
S = 1024
H = 16
KVH = 4
G = H // KVH
D = 128
WINDOW = 512
SCALE = 0.08838834764831845

BQ = 256
KS = BQ + WINDOW
NQ = S // BQ


def _slopes(n):
    def pow2(n):
        start = 2 ** (-(2 ** (-(math.log2(n) - 3))))
        return [start * start ** i for i in range(n)]
    if math.log2(n).is_integer():
        return pow2(n)
    closest = 2 ** math.floor(math.log2(n))
    return pow2(closest) + _slopes(2 * closest)[0::2][: n - closest]


def _attn_kernel(slopes_ref, q_ref, k_ref, v_ref, o_ref):
    h = pl.program_id(1)
    qi = pl.program_id(2)
    q_base = qi * BQ
    start = jnp.maximum(q_base + BQ - KS, 0)

    kspan = k_ref[0, pl.ds(start, KS), :]
    vspan = v_ref[0, pl.ds(start, KS), :]

    i_idx = q_base + jax.lax.broadcasted_iota(jnp.int32, (BQ, KS), 0)
    j_idx = start + jax.lax.broadcasted_iota(jnp.int32, (BQ, KS), 1)
    delta = (j_idx - i_idx).astype(jnp.float32)
    valid = (j_idx <= i_idx) & (j_idx >= i_idx - WINDOW)

    for g in range(G):
        qg = q_ref[0, :, g * D:(g + 1) * D]
        s = jax.lax.dot_general(
            qg, kspan, (((1,), (1,)), ((), ())),
            preferred_element_type=jnp.float32,
        ) * SCALE
        s = s + slopes_ref[h, g] * delta
        s = jnp.where(valid, s, jnp.float32(-1e30))
        m = jnp.max(s, axis=1, keepdims=True)
        p = jnp.exp(s - m)
        l = jnp.sum(p, axis=1, keepdims=True)
        p = p / l
        og = jax.lax.dot_general(
            p, vspan, (((1,), (0,)), ((), ())),
            preferred_element_type=jnp.float32,
        )
        o_ref[0, :, g * D:(g + 1) * D] = og


def kernel(q, k, v):
    qh = q.reshape(B, S, H * D)
    kh = k.reshape(B, S, KVH * D)
    vh = v.reshape(B, S, KVH * D)
    slopes = jnp.asarray(
        np.array(_slopes(H), dtype=np.float32).reshape(KVH, G))

    out = pl.pallas_call(
        _attn_kernel,
        grid=(B, KVH, NQ),
        in_specs=[
            pl.BlockSpec(memory_space=pltpu.SMEM),
            pl.BlockSpec((1, BQ, G * D), lambda b, h, qi: (b, qi, h)),
            pl.BlockSpec((1, S, D), lambda b, h, qi: (b, 0, h)),
            pl.BlockSpec((1, S, D), lambda b, h, qi: (b, 0, h)),
        ],
        out_specs=pl.BlockSpec((1, BQ, G * D), lambda b, h, qi: (b, qi, h)),
        out_shape=jax.ShapeDtypeStruct((B, S, H * D), jnp.float32),
    )(slopes, qh, kh, vh)
    return out.reshape(B * S, H * D)
